# baseline (speedup 1.0000x reference)
import jax
import jax.numpy as jnp
from jax import lax
from jax.experimental import pallas as pl
from jax.experimental.pallas import tpu as pltpu


def _complex_body(lhs_ref, rel_ref, out_ref):
    lhs = lhs_ref[...]
    rel = rel_ref[...]
    n, d = lhs.shape
    r = d // 2
    col = lax.broadcasted_iota(jnp.int32, (n, d), 1)
    first = col < r
    rrel = pltpu.roll(rel, r, 1)
    rlhs = pltpu.roll(lhs, r, 1)
    a = jnp.where(first, rel, rrel)
    b = jnp.where(first, -rrel, rel)
    out_ref[...] = lhs * a + rlhs * b


def kernel(lhs, rel):
    B, D = lhs.shape
    blk = 16384
    return pl.pallas_call(
        _complex_body,
        grid=(B // blk,),
        in_specs=[
            pl.BlockSpec((blk, D), lambda i: (i, 0)),
            pl.BlockSpec((blk, D), lambda i: (i, 0)),
        ],
        out_specs=pl.BlockSpec((blk, D), lambda i: (i, 0)),
        out_shape=jax.ShapeDtypeStruct((B, D), lhs.dtype),
    )(lhs, rel)
